# Initial kernel scaffold; baseline (speedup 1.0000x reference)
import jax
import jax.numpy as jnp
from jax.experimental import pallas as pl


def kernel(node_embeddings, node_type_embeddings, trans_weights, trans_weights_s1, trans_weights_s2, train_inputs, train_types, node_neigh):
    raise NotImplementedError("write your pallas kernel here")


# speedup vs baseline: 37.8718x; 37.8718x over previous
import functools

import jax
import jax.numpy as jnp
from jax import lax
from jax.experimental import pallas as pl
from jax.experimental.pallas import tpu as pltpu
from jax.experimental.pallas import tpu_sc as plsc

NUM_NODES = 100000
EMBED = 128
EMBED_U = 32
EDGE_TYPES = 4
DIM_A = 16
BATCH = 4096
NEIGH = 10

NC = 2
NS = 16
NW = NC * NS
G = BATCH * EDGE_TYPES
G_W = G // NW
CH = 64
NCH = G_W // CH
ROWS = CH * NEIGH
NSEG = ROWS // 128
BE_W = BATCH // NW

_BB = 512
_f32 = jnp.float32


def _sc_body(nembed_hbm, ntype_hbm, tin_hbm, nidx_hbm,
             ne_out, nte_out,
             tidx_v, trows_v, nidx_v, rows_v, acc_v, sem1, sem2):
    wid = lax.axis_index("s") * NC + lax.axis_index("c")

    b0 = wid * BE_W
    pltpu.sync_copy(tin_hbm.at[pl.ds(b0, BE_W)], tidx_v)
    pltpu.async_copy(nembed_hbm.at[tidx_v], trows_v, sem1).wait()
    pltpu.sync_copy(trows_v, ne_out.at[pl.ds(b0, BE_W)])

    g0 = wid * G_W
    nrows = G_W * NEIGH // 128
    pltpu.sync_copy(nidx_hbm.at[pl.ds(wid * nrows, nrows)], nidx_v)

    def chunk_body(c, carry):
        base = g0 + c * CH
        cps = []
        for s in range(NSEG):
            cps.append(pltpu.async_copy(ntype_hbm.at[nidx_v.at[c * NSEG + s]],
                                        rows_v.at[pl.ds(s * 128, 128)], sem2))
        for cp in cps:
            cp.wait()

        def grp(j, carry2):
            r0 = j * NEIGH
            for h in (0, 16):
                a = rows_v[r0, pl.ds(h, 16)]
                for n in range(1, NEIGH):
                    a = a + rows_v[r0 + n, pl.ds(h, 16)]
                acc_v[j, pl.ds(h, 16)] = a
            return carry2

        lax.fori_loop(0, CH, grp, 0, unroll=2)
        pltpu.sync_copy(acc_v, nte_out.at[pl.ds(base, CH)])
        return carry

    lax.fori_loop(0, NCH, chunk_body, 0)


@functools.partial(jax.jit, static_argnums=())
def _sc_gather(node_embeddings, ntype_flat, train_inputs, nidx2d):
    mesh = plsc.VectorSubcoreMesh(core_axis_name="c", subcore_axis_name="s",
                                  num_cores=NC, num_subcores=NS)
    f = pl.kernel(
        _sc_body,
        out_type=(jax.ShapeDtypeStruct((BATCH, EMBED), _f32),
                  jax.ShapeDtypeStruct((G, EMBED_U), _f32)),
        mesh=mesh,
        scratch_types=[
            pltpu.VMEM((BE_W,), jnp.int32),
            pltpu.VMEM((BE_W, EMBED), _f32),
            pltpu.VMEM((G_W * NEIGH // 128, 128), jnp.int32),
            pltpu.VMEM((ROWS, EMBED_U), _f32),
            pltpu.VMEM((CH, EMBED_U), _f32),
            pltpu.SemaphoreType.DMA,
            pltpu.SemaphoreType.DMA,
        ],
        compiler_params=pltpu.CompilerParams(use_tc_tiling_on_sc=False),
    )
    return f(node_embeddings, ntype_flat, train_inputs, nidx2d)


def _dense_body(types_ref, ne_ref, nte_ref, s1_ref, s2_ref, w_ref, out_ref):
    types = types_ref[0, 0, :]
    x = nte_ref[...]
    ctype = lax.broadcasted_iota(jnp.int32, (_BB, EMBED), 1) // EMBED_U
    colmask = (types[:, None] == ctype).astype(_f32)
    oh8 = (types[:, None] == lax.broadcasted_iota(jnp.int32, (_BB, 8), 1)
           ).astype(_f32)
    s2sel = jnp.dot(oh8, s2_ref[...], preferred_element_type=_f32)
    s1 = s1_ref[...]

    scs = []
    for t in range(EDGE_TYPES):
        xt = x[:, t * EMBED_U:(t + 1) * EMBED_U]
        xt4 = jnp.concatenate([xt] * EDGE_TYPES, axis=1) * colmask
        h = jnp.tanh(jnp.dot(xt4, s1, preferred_element_type=_f32))
        scs.append(jnp.sum(h * s2sel, axis=1, keepdims=True))
    scores = jnp.concatenate(scs, axis=1)
    m = jnp.max(scores, axis=1, keepdims=True)
    e = jnp.exp(scores - m)
    att = e / jnp.sum(e, axis=1, keepdims=True)

    comb = att[:, 0:1] * x[:, 0:EMBED_U]
    for t in range(1, EDGE_TYPES):
        comb = comb + att[:, t:t + 1] * x[:, t * EMBED_U:(t + 1) * EMBED_U]
    comb4 = jnp.concatenate([comb] * EDGE_TYPES, axis=1) * colmask
    out = ne_ref[...] + jnp.dot(comb4, w_ref[...], preferred_element_type=_f32)
    nrm = jnp.sqrt(jnp.sum(out * out, axis=1, keepdims=True))
    out_ref[...] = out / jnp.maximum(nrm, 1e-12)


def _dense(types3d, ne_g, nte_flat, s1f, s2p, wf):
    grid = (BATCH // _BB,)
    return pl.pallas_call(
        _dense_body,
        grid=grid,
        in_specs=[
            pl.BlockSpec((1, 1, _BB), lambda i: (i, 0, 0)),
            pl.BlockSpec((_BB, EMBED), lambda i: (i, 0)),
            pl.BlockSpec((_BB, EMBED), lambda i: (i, 0)),
            pl.BlockSpec((EMBED, DIM_A), lambda i: (0, 0)),
            pl.BlockSpec((8, DIM_A), lambda i: (0, 0)),
            pl.BlockSpec((EMBED, EMBED), lambda i: (0, 0)),
        ],
        out_specs=pl.BlockSpec((_BB, EMBED), lambda i: (i, 0)),
        out_shape=jax.ShapeDtypeStruct((BATCH, EMBED), _f32),
    )(types3d, ne_g, nte_flat, s1f, s2p, wf)


def kernel(node_embeddings, node_type_embeddings, trans_weights,
           trans_weights_s1, trans_weights_s2, train_inputs, train_types,
           node_neigh):
    ntype_flat = node_type_embeddings.reshape(NUM_NODES * EDGE_TYPES, EMBED_U)
    tin = train_inputs.astype(jnp.int32)
    flat_idx = (node_neigh.astype(jnp.int32) * EDGE_TYPES
                + jnp.arange(EDGE_TYPES, dtype=jnp.int32)[None, :, None])
    nidx2d = flat_idx.reshape(G * NEIGH // 128, 128)

    ne_g, nte_sum = _sc_gather(node_embeddings, ntype_flat, tin, nidx2d)

    types3d = train_types.astype(jnp.int32).reshape(BATCH // _BB, 1, _BB)
    nte_flat = nte_sum.reshape(BATCH, EDGE_TYPES * EMBED_U)
    s1f = trans_weights_s1.reshape(EDGE_TYPES * EMBED_U, DIM_A)
    s2r = trans_weights_s2.reshape(EDGE_TYPES, DIM_A)
    s2p = jnp.concatenate([s2r, jnp.zeros((4, DIM_A), _f32)], axis=0)
    wf = trans_weights.reshape(EDGE_TYPES * EMBED_U, EMBED)
    return _dense(types3d, ne_g, nte_flat, s1f, s2p, wf)


# speedup vs baseline: 61.8333x; 1.6327x over previous
import functools

import jax
import jax.numpy as jnp
from jax import lax
from jax.experimental import pallas as pl
from jax.experimental.pallas import tpu as pltpu
from jax.experimental.pallas import tpu_sc as plsc

NUM_NODES = 100000
EMBED = 128
EMBED_U = 32
EDGE_TYPES = 4
DIM_A = 16
BATCH = 4096
NEIGH = 10

NC = 2
NS = 16
NW = NC * NS
G = BATCH * EDGE_TYPES
G_W = G // NW
CH = 64
NCH = G_W // CH
ROWS = CH * NEIGH
NSEG = ROWS // 128
BE_W = BATCH // NW
B_CH = CH // EDGE_TYPES

_BB = 512
_f32 = jnp.float32


def _sc_body(nembed_hbm, ntype_hbm, tin_hbm, nidx_hbm,
             ne_out, nte_out,
             tidx_v, trows_v, nidx_v, rows_v, acc_v, sem1, sem2):
    wid = lax.axis_index("s") * NC + lax.axis_index("c")

    b0 = wid * BE_W
    pltpu.sync_copy(tin_hbm.at[pl.ds(b0, BE_W)], tidx_v)
    ne_cp = pltpu.async_copy(nembed_hbm.at[tidx_v], trows_v, sem1)

    nrows = G_W * NEIGH // 128
    pltpu.sync_copy(nidx_hbm.at[pl.ds(wid * nrows, nrows)], nidx_v)

    ne_cp.wait()
    pltpu.sync_copy(trows_v, ne_out.at[pl.ds(b0, BE_W)])

    def chunk_body(c, carry):
        cps = []
        for s in range(NSEG):
            cps.append(pltpu.async_copy(ntype_hbm.at[nidx_v.at[c * NSEG + s]],
                                        rows_v.at[pl.ds(s * 128, 128)], sem2))
        for cp in cps:
            cp.wait()

        def grp(b_loc, carry2):
            r0 = b_loc * EDGE_TYPES * NEIGH
            for t in range(EDGE_TYPES):
                rt = r0 + t * NEIGH
                for h in (0, 16):
                    col = t * EMBED_U + h
                    a = rows_v[rt, pl.ds(col, 16)]
                    for n in range(1, NEIGH):
                        a = a + rows_v[rt + n, pl.ds(col, 16)]
                    acc_v[b_loc, pl.ds(col, 16)] = a
            return carry2

        lax.fori_loop(0, B_CH, grp, 0)
        pltpu.sync_copy(acc_v,
                        nte_out.at[pl.ds(wid * (G_W // EDGE_TYPES) + c * B_CH,
                                         B_CH)])
        return carry

    lax.fori_loop(0, NCH, chunk_body, 0)


def _sc_gather(node_embeddings, ntype_rows, train_inputs, nidx2d):
    mesh = plsc.VectorSubcoreMesh(core_axis_name="c", subcore_axis_name="s",
                                  num_cores=NC, num_subcores=NS)
    f = pl.kernel(
        _sc_body,
        out_type=(jax.ShapeDtypeStruct((BATCH, EMBED), _f32),
                  jax.ShapeDtypeStruct((BATCH, EMBED), _f32)),
        mesh=mesh,
        scratch_types=[
            pltpu.VMEM((BE_W,), jnp.int32),
            pltpu.VMEM((BE_W, EMBED), _f32),
            pltpu.VMEM((G_W * NEIGH // 128, 128), jnp.int32),
            pltpu.VMEM((ROWS, EMBED), _f32),
            pltpu.VMEM((B_CH, EMBED), _f32),
            pltpu.SemaphoreType.DMA,
            pltpu.SemaphoreType.DMA,
        ],
    )
    return f(node_embeddings, ntype_rows, train_inputs, nidx2d)


def _dense_body(types_ref, ne_ref, nte_ref, s1_ref, s2_ref, w_ref, out_ref):
    types = types_ref[0, 0, :]
    x = nte_ref[...]
    ctype = lax.broadcasted_iota(jnp.int32, (_BB, EMBED), 1) // EMBED_U
    colmask = (types[:, None] == ctype).astype(_f32)
    oh8 = (types[:, None] == lax.broadcasted_iota(jnp.int32, (_BB, 8), 1)
           ).astype(_f32)
    s2sel = jnp.dot(oh8, s2_ref[...], preferred_element_type=_f32)
    s1 = s1_ref[...]

    scs = []
    for t in range(EDGE_TYPES):
        xt = x[:, t * EMBED_U:(t + 1) * EMBED_U]
        xt4 = jnp.concatenate([xt] * EDGE_TYPES, axis=1) * colmask
        h = jnp.tanh(jnp.dot(xt4, s1, preferred_element_type=_f32))
        scs.append(jnp.sum(h * s2sel, axis=1, keepdims=True))
    scores = jnp.concatenate(scs, axis=1)
    m = jnp.max(scores, axis=1, keepdims=True)
    e = jnp.exp(scores - m)
    att = e / jnp.sum(e, axis=1, keepdims=True)

    comb = att[:, 0:1] * x[:, 0:EMBED_U]
    for t in range(1, EDGE_TYPES):
        comb = comb + att[:, t:t + 1] * x[:, t * EMBED_U:(t + 1) * EMBED_U]
    comb4 = jnp.concatenate([comb] * EDGE_TYPES, axis=1) * colmask
    out = ne_ref[...] + jnp.dot(comb4, w_ref[...], preferred_element_type=_f32)
    nrm = jnp.sqrt(jnp.sum(out * out, axis=1, keepdims=True))
    out_ref[...] = out / jnp.maximum(nrm, 1e-12)


def _dense(types3d, ne_g, nte, s1f, s2p, wf):
    grid = (BATCH // _BB,)
    return pl.pallas_call(
        _dense_body,
        grid=grid,
        in_specs=[
            pl.BlockSpec((1, 1, _BB), lambda i: (i, 0, 0)),
            pl.BlockSpec((_BB, EMBED), lambda i: (i, 0)),
            pl.BlockSpec((_BB, EMBED), lambda i: (i, 0)),
            pl.BlockSpec((EMBED, DIM_A), lambda i: (0, 0)),
            pl.BlockSpec((8, DIM_A), lambda i: (0, 0)),
            pl.BlockSpec((EMBED, EMBED), lambda i: (0, 0)),
        ],
        out_specs=pl.BlockSpec((_BB, EMBED), lambda i: (i, 0)),
        out_shape=jax.ShapeDtypeStruct((BATCH, EMBED), _f32),
    )(types3d, ne_g, nte, s1f, s2p, wf)


def kernel(node_embeddings, node_type_embeddings, trans_weights,
           trans_weights_s1, trans_weights_s2, train_inputs, train_types,
           node_neigh):
    ntype_rows = node_type_embeddings.reshape(NUM_NODES, EDGE_TYPES * EMBED_U)
    tin = train_inputs.astype(jnp.int32)
    nidx2d = node_neigh.astype(jnp.int32).reshape(G * NEIGH // 128, 128)

    ne_g, nte = _sc_gather(node_embeddings, ntype_rows, tin, nidx2d)

    types3d = train_types.astype(jnp.int32).reshape(BATCH // _BB, 1, _BB)
    s1f = trans_weights_s1.reshape(EDGE_TYPES * EMBED_U, DIM_A)
    s2r = trans_weights_s2.reshape(EDGE_TYPES, DIM_A)
    s2p = jnp.concatenate([s2r, jnp.zeros((4, DIM_A), _f32)], axis=0)
    wf = trans_weights.reshape(EDGE_TYPES * EMBED_U, EMBED)
    return _dense(types3d, ne_g, nte, s1f, s2p, wf)
